# baseline (speedup 1.0000x reference)
import jax
import jax.numpy as jnp
from jax.experimental import pallas as pl

NUM_CLASSES = 1000
CTX_LEN = 4
EMB_DIM = 64
ROW = (CTX_LEN + 1) * EMB_DIM
CTX_FLAT = CTX_LEN * EMB_DIM
BLOCK_ROWS = 40


def _body(ctx_ref, emb_ref, out_ref):
    ctx = ctx_ref[...]
    emb = emb_ref[...]
    bc = jnp.broadcast_to(ctx, (BLOCK_ROWS, CTX_FLAT))
    out_ref[...] = jnp.concatenate([bc, emb], axis=1)


def kernel(context, emb_table):
    ctx2 = context.reshape(1, CTX_FLAT)
    out2d = pl.pallas_call(
        _body,
        grid=(NUM_CLASSES // BLOCK_ROWS,),
        in_specs=[
            pl.BlockSpec((1, CTX_FLAT), lambda i: (0, 0)),
            pl.BlockSpec((BLOCK_ROWS, EMB_DIM), lambda i: (i, 0)),
        ],
        out_specs=pl.BlockSpec((BLOCK_ROWS, ROW), lambda i: (i, 0)),
        out_shape=jax.ShapeDtypeStruct((NUM_CLASSES, ROW), jnp.float32),
    )(ctx2, emb_table)
    return out2d.reshape(NUM_CLASSES, CTX_LEN + 1, EMB_DIM)


# speedup vs baseline: 2.2350x; 2.2350x over previous
import jax
import jax.numpy as jnp
from jax.experimental import pallas as pl

NUM_CLASSES = 1000
CTX_LEN = 4
EMB_DIM = 64
ROW = (CTX_LEN + 1) * EMB_DIM
CTX_FLAT = CTX_LEN * EMB_DIM
BLOCK_ROWS = 1000


def _body(ctx_ref, emb_ref, out_ref):
    ctx = ctx_ref[...]
    emb = emb_ref[...]
    bc = jnp.broadcast_to(ctx, (BLOCK_ROWS, CTX_FLAT))
    out_ref[...] = jnp.concatenate([bc, emb], axis=1)


def kernel(context, emb_table):
    ctx2 = context.reshape(1, CTX_FLAT)
    out2d = pl.pallas_call(
        _body,
        grid=(NUM_CLASSES // BLOCK_ROWS,),
        in_specs=[
            pl.BlockSpec((1, CTX_FLAT), lambda i: (0, 0)),
            pl.BlockSpec((BLOCK_ROWS, EMB_DIM), lambda i: (i, 0)),
        ],
        out_specs=pl.BlockSpec((BLOCK_ROWS, ROW), lambda i: (i, 0)),
        out_shape=jax.ShapeDtypeStruct((NUM_CLASSES, ROW), jnp.float32),
    )(ctx2, emb_table)
    return out2d.reshape(NUM_CLASSES, CTX_LEN + 1, EMB_DIM)
